# Initial kernel scaffold; baseline (speedup 1.0000x reference)
import jax
import jax.numpy as jnp
from jax.experimental import pallas as pl


def kernel(t, embed_table):
    raise NotImplementedError("write your pallas kernel here")


# speedup vs baseline: 2.0763x; 2.0763x over previous
import functools

import jax
import jax.numpy as jnp
import numpy as np
from jax import lax
from jax.experimental import pallas as pl
from jax.experimental.pallas import tpu as pltpu
from jax.experimental.pallas import tpu_sc as plsc

MAX_FRAMES = 1024
D = 256
NUM_FEATS = 10
HALF = D + 2 * NUM_FEATS
CHUNK = 64
LUT_N = 2 * MAX_FRAMES + 512

_SIN_LUT = np.sin(np.pi * np.arange(LUT_N, dtype=np.float64) / MAX_FRAMES)
_SIN_LUT = _SIN_LUT.astype(np.float32)


def _make_kernel(B: int, n_workers: int):
    rows_per_w = B // n_workers
    n_chunks = rows_per_w // CHUNK
    assert rows_per_w % CHUNK == 0

    mesh = plsc.VectorSubcoreMesh(core_axis_name="c", subcore_axis_name="s")
    nc = plsc.get_sparse_core_info().num_cores

    @functools.partial(
        pl.kernel,
        mesh=mesh,
        out_type=jax.ShapeDtypeStruct((2 * B, HALF), jnp.float32),
        compiler_params=pltpu.CompilerParams(needs_layout_passes=False),
        scratch_types=[
            pltpu.VMEM((3 * rows_per_w,), jnp.int32),
            pltpu.VMEM((LUT_N,), jnp.float32),
            pltpu.VMEM((2 * CHUNK,), jnp.int32),
            pltpu.VMEM((2 * CHUNK, HALF), jnp.float32),
            pltpu.SemaphoreType.DMA,
        ],
    )
    def enc(t_hbm, table_hbm, lut_hbm, out_hbm, tbuf, lutbuf, idxb, buf, sem):
        wid = lax.axis_index("s") * nc + lax.axis_index("c")
        base_row = wid * rows_per_w

        pltpu.sync_copy(lut_hbm, lutbuf)
        pltpu.sync_copy(t_hbm.at[pl.ds(base_row * 3, rows_per_w * 3)], tbuf)

        lane = lax.iota(jnp.int32, 16)

        for ch in range(n_chunks):
            ch_off = ch * CHUNK

            def diff_body(g, _):
                r = g * 16
                f = 3 * (ch_off + r) + 3 * lane
                a = plsc.load_gather(tbuf, [f])
                b = plsc.load_gather(tbuf, [f + 1])
                c = plsc.load_gather(tbuf, [f + 2])
                pos = 2 * (r + lane)
                plsc.store_scatter(idxb, [pos], b - a)
                plsc.store_scatter(idxb, [pos + 1], c - b)
                return 0

            lax.fori_loop(0, CHUNK // 16, diff_body, 0)

            cp = pltpu.async_copy(
                table_hbm.at[idxb], buf.at[:, pl.ds(0, D)], sem)

            def four_body(g, _):
                r = g * 16
                rows16 = r + lane
                d = idxb[pl.ds(r, 16)]
                for k in range(NUM_FEATS):
                    m = (d << k) & (2 * MAX_FRAMES - 1)
                    s = plsc.load_gather(lutbuf, [m])
                    c = plsc.load_gather(lutbuf, [m + 512])
                    col = jnp.full((16,), D + k, dtype=jnp.int32)
                    plsc.store_scatter(buf, [rows16, col], s)
                    plsc.store_scatter(buf, [rows16, col + NUM_FEATS], c)
                return 0

            lax.fori_loop(0, 2 * CHUNK // 16, four_body, 0)

            cp.wait()
            pltpu.sync_copy(
                buf, out_hbm.at[pl.ds(2 * (base_row + ch_off), 2 * CHUNK)])

    return enc


def kernel(t, embed_table):
    B = t.shape[0]
    t_flat = t.reshape(-1).astype(jnp.int32)
    lut = jnp.asarray(_SIN_LUT)
    enc = _make_kernel(B, 32)
    out2 = enc(t_flat, embed_table, lut)
    return out2.reshape(B, 2 * HALF)


# speedup vs baseline: 2.1118x; 1.0171x over previous
import functools

import jax
import jax.numpy as jnp
import numpy as np
from jax import lax
from jax.experimental import pallas as pl
from jax.experimental.pallas import tpu as pltpu
from jax.experimental.pallas import tpu_sc as plsc

MAX_FRAMES = 1024
D = 256
NUM_FEATS = 10
HALF = D + 2 * NUM_FEATS
CHUNK = 64
LUT_N = 2 * MAX_FRAMES + 512

_SIN_LUT = np.sin(np.pi * np.arange(LUT_N, dtype=np.float64) / MAX_FRAMES)
_SIN_LUT = _SIN_LUT.astype(np.float32)


def _make_kernel(B: int, n_workers: int):
    rows_per_w = B // n_workers
    n_chunks = rows_per_w // CHUNK
    assert rows_per_w % CHUNK == 0

    mesh = plsc.VectorSubcoreMesh(core_axis_name="c", subcore_axis_name="s")
    nc = plsc.get_sparse_core_info().num_cores

    @functools.partial(
        pl.kernel,
        mesh=mesh,
        out_type=jax.ShapeDtypeStruct((2 * B, HALF), jnp.float32),
        compiler_params=pltpu.CompilerParams(needs_layout_passes=False),
        scratch_types=[
            pltpu.VMEM((3 * rows_per_w,), jnp.int32),
            pltpu.VMEM((LUT_N,), jnp.float32),
            pltpu.VMEM((2 * CHUNK,), jnp.int32),
            pltpu.VMEM((2 * CHUNK,), jnp.int32),
            pltpu.VMEM((2 * CHUNK, HALF), jnp.float32),
            pltpu.VMEM((2 * CHUNK, HALF), jnp.float32),
            pltpu.SemaphoreType.DMA,
            pltpu.SemaphoreType.DMA,
            pltpu.SemaphoreType.DMA,
            pltpu.SemaphoreType.DMA,
        ],
    )
    def enc(t_hbm, table_hbm, lut_hbm, out_hbm, tbuf, lutbuf, idxb0, idxb1,
            buf0, buf1, semg0, semg1, semo0, semo1):
        wid = lax.axis_index("s") * nc + lax.axis_index("c")
        base_row = wid * rows_per_w

        pltpu.sync_copy(lut_hbm, lutbuf)
        pltpu.sync_copy(t_hbm.at[pl.ds(base_row * 3, rows_per_w * 3)], tbuf)

        lane = lax.iota(jnp.int32, 16)
        idxbs = (idxb0, idxb1)
        bufs = (buf0, buf1)
        semgs = (semg0, semg1)
        semos = (semo0, semo1)
        out_cps = [None, None]

        for ch in range(n_chunks):
            p = ch % 2
            idxb, buf = idxbs[p], bufs[p]
            ch_off = ch * CHUNK

            if out_cps[p] is not None:
                out_cps[p].wait()

            def diff_body(g, _, idxb=idxb, ch_off=ch_off):
                r = g * 16
                f = 3 * (ch_off + r) + 3 * lane
                a = plsc.load_gather(tbuf, [f])
                b = plsc.load_gather(tbuf, [f + 1])
                c = plsc.load_gather(tbuf, [f + 2])
                pos = 2 * (r + lane)
                plsc.store_scatter(idxb, [pos], b - a)
                plsc.store_scatter(idxb, [pos + 1], c - b)
                return 0

            lax.fori_loop(0, CHUNK // 16, diff_body, 0)

            cp = pltpu.async_copy(
                table_hbm.at[idxb], buf.at[:, pl.ds(0, D)], semgs[p])

            def four_body(g, _, idxb=idxb, buf=buf):
                r = g * 16
                rows16 = r + lane
                d = idxb[pl.ds(r, 16)]
                for k in range(NUM_FEATS):
                    m = (d << k) & (2 * MAX_FRAMES - 1)
                    s = plsc.load_gather(lutbuf, [m])
                    c = plsc.load_gather(lutbuf, [m + 512])
                    col = jnp.full((16,), D + k, dtype=jnp.int32)
                    plsc.store_scatter(buf, [rows16, col], s)
                    plsc.store_scatter(buf, [rows16, col + NUM_FEATS], c)
                return 0

            lax.fori_loop(0, 2 * CHUNK // 16, four_body, 0)

            cp.wait()
            out_cps[p] = pltpu.async_copy(
                buf, out_hbm.at[pl.ds(2 * (base_row + ch_off), 2 * CHUNK)],
                semos[p])

        out_cps[0].wait()
        out_cps[1].wait()

    return enc


def kernel(t, embed_table):
    B = t.shape[0]
    t_flat = t.reshape(-1).astype(jnp.int32)
    lut = jnp.asarray(_SIN_LUT)
    enc = _make_kernel(B, 32)
    out2 = enc(t_flat, embed_table, lut)
    return out2.reshape(B, 2 * HALF)


# speedup vs baseline: 2.5628x; 1.2136x over previous
import functools

import jax
import jax.numpy as jnp
import numpy as np
from jax import lax
from jax.experimental import pallas as pl
from jax.experimental.pallas import tpu as pltpu
from jax.experimental.pallas import tpu_sc as plsc

MAX_FRAMES = 1024
D = 256
NUM_FEATS = 10
HALF = D + 2 * NUM_FEATS
ROW = 2 * HALF
CHUNK = 64
LUT_N = 2 * MAX_FRAMES + 512

_SIN_LUT = np.sin(np.pi * np.arange(LUT_N, dtype=np.float64) / MAX_FRAMES)
_SIN_LUT = _SIN_LUT.astype(np.float32)


def _make_kernel(B: int, n_workers: int):
    rows_per_w = B // n_workers
    n_chunks = rows_per_w // CHUNK
    assert rows_per_w % CHUNK == 0

    mesh = plsc.VectorSubcoreMesh(core_axis_name="c", subcore_axis_name="s")
    nc = plsc.get_sparse_core_info().num_cores

    @functools.partial(
        pl.kernel,
        mesh=mesh,
        out_type=jax.ShapeDtypeStruct((B, ROW), jnp.float32),
        compiler_params=pltpu.CompilerParams(needs_layout_passes=False),
        scratch_types=[
            pltpu.VMEM((3 * rows_per_w,), jnp.int32),
            pltpu.VMEM((LUT_N,), jnp.float32),
            pltpu.VMEM((CHUNK,), jnp.int32),
            pltpu.VMEM((CHUNK,), jnp.int32),
            pltpu.VMEM((CHUNK,), jnp.int32),
            pltpu.VMEM((CHUNK,), jnp.int32),
            pltpu.VMEM((CHUNK, ROW), jnp.float32),
            pltpu.VMEM((CHUNK, ROW), jnp.float32),
            pltpu.VMEM((CHUNK, D), jnp.float32),
            pltpu.VMEM((CHUNK, D), jnp.float32),
            pltpu.SemaphoreType.DMA,
            pltpu.SemaphoreType.DMA,
            pltpu.SemaphoreType.DMA,
            pltpu.SemaphoreType.DMA,
            pltpu.SemaphoreType.DMA,
            pltpu.SemaphoreType.DMA,
        ],
    )
    def enc(t_hbm, table_hbm, lut_hbm, out_hbm, tbuf, lutbuf, ia0, ia1,
            ib0, ib1, buf0, buf1, st0, st1, sa0, sa1, sb0, sb1, so0, so1):
        wid = lax.axis_index("s") * nc + lax.axis_index("c")
        base_row = wid * rows_per_w

        pltpu.sync_copy(lut_hbm, lutbuf)
        pltpu.sync_copy(t_hbm.at[pl.ds(base_row * 3, rows_per_w * 3)], tbuf)

        lane = lax.iota(jnp.int32, 16)
        ias = (ia0, ia1)
        ibs = (ib0, ib1)
        bufs = (buf0, buf1)
        sts = (st0, st1)
        sas = (sa0, sa1)
        sbs = (sb0, sb1)
        sos = (so0, so1)
        out_cps = [None, None]

        for ch in range(n_chunks):
            p = ch % 2
            ia, ib, buf, st = ias[p], ibs[p], bufs[p], sts[p]
            ch_off = ch * CHUNK

            if out_cps[p] is not None:
                out_cps[p].wait()

            def diff_body(g, _, ia=ia, ib=ib, ch_off=ch_off):
                r = g * 16
                f = 3 * (ch_off + r) + 3 * lane
                a = plsc.load_gather(tbuf, [f])
                b = plsc.load_gather(tbuf, [f + 1])
                c = plsc.load_gather(tbuf, [f + 2])
                ia[pl.ds(r, 16)] = b - a
                ib[pl.ds(r, 16)] = c - b
                return 0

            lax.fori_loop(0, CHUNK // 16, diff_body, 0)

            cpa = pltpu.async_copy(
                table_hbm.at[ia], buf.at[:, pl.ds(0, D)], sas[p])
            cpb = pltpu.async_copy(table_hbm.at[ib], st, sbs[p])

            def four_body(g, _, ia=ia, ib=ib, buf=buf):
                r = g * 16
                rows16 = r + lane
                d0 = ia[pl.ds(r, 16)]
                d1 = ib[pl.ds(r, 16)]
                for k in range(NUM_FEATS):
                    m0 = (d0 << k) & (2 * MAX_FRAMES - 1)
                    m1 = (d1 << k) & (2 * MAX_FRAMES - 1)
                    col = jnp.full((16,), D + k, dtype=jnp.int32)
                    plsc.store_scatter(
                        buf, [rows16, col], plsc.load_gather(lutbuf, [m0]))
                    plsc.store_scatter(
                        buf, [rows16, col + NUM_FEATS],
                        plsc.load_gather(lutbuf, [m0 + 512]))
                    plsc.store_scatter(
                        buf, [rows16, col + HALF],
                        plsc.load_gather(lutbuf, [m1]))
                    plsc.store_scatter(
                        buf, [rows16, col + HALF + NUM_FEATS],
                        plsc.load_gather(lutbuf, [m1 + 512]))
                return 0

            lax.fori_loop(0, CHUNK // 16, four_body, 0)

            cpb.wait()

            def reloc_body(r, _, buf=buf, st=st):
                rr = jnp.full((16,), r, dtype=jnp.int32)
                for i in range(D // 16):
                    v = st[r, pl.ds(16 * i, 16)]
                    plsc.store_scatter(buf, [rr, HALF + 16 * i + lane], v)
                return 0

            lax.fori_loop(0, CHUNK, reloc_body, 0)

            cpa.wait()
            out_cps[p] = pltpu.async_copy(
                buf, out_hbm.at[pl.ds(base_row + ch_off, CHUNK)], sos[p])

        out_cps[0].wait()
        out_cps[1].wait()

    return enc


def kernel(t, embed_table):
    B = t.shape[0]
    t_flat = t.reshape(-1).astype(jnp.int32)
    lut = jnp.asarray(_SIN_LUT)
    enc = _make_kernel(B, 32)
    return enc(t_flat, embed_table, lut)
